# baseline (speedup 1.0000x reference)
import jax
import jax.numpy as jnp
from jax.experimental import pallas as pl
from jax.experimental.pallas import tpu as pltpu

_N_GROUP = 2
_GROUP_SIZE = 4
_N_EXPERTS = _N_GROUP * _GROUP_SIZE
_ROUTED_SCALING = 2.5

_BLOCK_N = 1024
_OUT_W = 128


def _gate_kernel(x_ref, w_ref, out_ref):
    x = x_ref[...]
    w = w_ref[...]
    logits = jax.lax.dot_general(
        x, w, (((1,), (1,)), ((), ())), preferred_element_type=jnp.float32
    )

    col = jax.lax.broadcasted_iota(jnp.int32, logits.shape, 1)
    in_g0 = col < _GROUP_SIZE
    neg = jnp.float32(-jnp.inf)
    m0 = jnp.max(jnp.where(in_g0, logits, neg), axis=1, keepdims=True)
    m1 = jnp.max(jnp.where(in_g0, neg, logits), axis=1, keepdims=True)
    big = jnp.int32(_N_EXPERTS)
    i0 = jnp.min(jnp.where(in_g0 & (logits >= m0), col, big),
                 axis=1, keepdims=True)
    i1 = jnp.min(jnp.where((~in_g0) & (logits >= m1), col, big),
                 axis=1, keepdims=True)
    s0 = jax.nn.sigmoid(m0)
    s1 = jax.nn.sigmoid(m1)
    inv = _ROUTED_SCALING / (s0 + s1 + 1e-10)
    pad = jnp.zeros((logits.shape[0], _OUT_W - _N_EXPERTS - 2 * _N_GROUP),
                    jnp.float32)
    out_ref[...] = jnp.concatenate(
        [logits, s0 * inv, s1 * inv,
         i0.astype(jnp.float32), i1.astype(jnp.float32), pad],
        axis=1,
    )


def kernel(hidden_states, gate_weight):
    n, d = hidden_states.shape
    e = gate_weight.shape[0]
    packed = pl.pallas_call(
        _gate_kernel,
        grid=(n // _BLOCK_N,),
        in_specs=[
            pl.BlockSpec((_BLOCK_N, d), lambda i: (i, 0)),
            pl.BlockSpec((e, d), lambda i: (0, 0)),
        ],
        out_specs=pl.BlockSpec((_BLOCK_N, _OUT_W), lambda i: (i, 0)),
        out_shape=jax.ShapeDtypeStruct((n, _OUT_W), jnp.float32),
        compiler_params=pltpu.CompilerParams(
            dimension_semantics=("parallel",),
        ),
    )(hidden_states, gate_weight)
    gate_logits = packed[:, :_N_EXPERTS]
    topk_weight = packed[:, _N_EXPERTS:_N_EXPERTS + _N_GROUP]
    topk_idx = packed[:, _N_EXPERTS + _N_GROUP:
                      _N_EXPERTS + 2 * _N_GROUP].astype(jnp.int32)
    return (topk_idx, topk_weight, gate_logits)


# speedup vs baseline: 1.4458x; 1.4458x over previous
import jax
import jax.numpy as jnp
from jax.experimental import pallas as pl
from jax.experimental.pallas import tpu as pltpu

_N_GROUP = 2
_GROUP_SIZE = 4
_N_EXPERTS = _N_GROUP * _GROUP_SIZE
_ROUTED_SCALING = 2.5

_BLOCK_N = 1024


def _gate_kernel(x_ref, w_ref, logits_t_ref, idx_t_ref, wgt_t_ref):
    x = x_ref[...]
    w = w_ref[...]
    logits = jax.lax.dot_general(
        x, w, (((1,), (1,)), ((), ())), preferred_element_type=jnp.float32
    )
    logits_t_ref[...] = logits.T

    col = jax.lax.broadcasted_iota(jnp.int32, logits.shape, 1)
    in_g0 = col < _GROUP_SIZE
    neg = jnp.float32(-jnp.inf)
    m0 = jnp.max(jnp.where(in_g0, logits, neg), axis=1, keepdims=True)
    m1 = jnp.max(jnp.where(in_g0, neg, logits), axis=1, keepdims=True)
    big = jnp.int32(_N_EXPERTS)
    i0 = jnp.min(jnp.where(in_g0 & (logits >= m0), col, big),
                 axis=1, keepdims=True)
    i1 = jnp.min(jnp.where((~in_g0) & (logits >= m1), col, big),
                 axis=1, keepdims=True)
    s0 = jax.nn.sigmoid(m0)
    s1 = jax.nn.sigmoid(m1)
    inv = _ROUTED_SCALING / (s0 + s1 + 1e-10)
    idx_t_ref[...] = jnp.concatenate([i0, i1], axis=1).T
    wgt_t_ref[...] = jnp.concatenate([s0 * inv, s1 * inv], axis=1).T


def kernel(hidden_states, gate_weight):
    n, d = hidden_states.shape
    e = gate_weight.shape[0]
    logits_t, idx_t, wgt_t = pl.pallas_call(
        _gate_kernel,
        grid=(n // _BLOCK_N,),
        in_specs=[
            pl.BlockSpec((_BLOCK_N, d), lambda i: (i, 0)),
            pl.BlockSpec((e, d), lambda i: (0, 0)),
        ],
        out_specs=[
            pl.BlockSpec((e, _BLOCK_N), lambda i: (0, i)),
            pl.BlockSpec((_N_GROUP, _BLOCK_N), lambda i: (0, i)),
            pl.BlockSpec((_N_GROUP, _BLOCK_N), lambda i: (0, i)),
        ],
        out_shape=[
            jax.ShapeDtypeStruct((e, n), jnp.float32),
            jax.ShapeDtypeStruct((_N_GROUP, n), jnp.int32),
            jax.ShapeDtypeStruct((_N_GROUP, n), jnp.float32),
        ],
        compiler_params=pltpu.CompilerParams(
            dimension_semantics=("parallel",),
        ),
    )(hidden_states, gate_weight)
    return (idx_t.T, wgt_t.T, logits_t.T)


# speedup vs baseline: 1.8214x; 1.2598x over previous
import jax
import jax.numpy as jnp
from jax.experimental import pallas as pl
from jax.experimental.pallas import tpu as pltpu

_N_GROUP = 2
_GROUP_SIZE = 4
_N_EXPERTS = _N_GROUP * _GROUP_SIZE
_ROUTED_SCALING = 2.5

_BLOCK_N = 1024


def _gate_kernel(x_ref, w_ref, logits_t_ref, idx_t_ref, wgt_t_ref):
    x = x_ref[...]
    w = w_ref[...]
    logits = jax.lax.dot_general(
        x, w, (((1,), (1,)), ((), ())), preferred_element_type=jnp.float32
    )
    lt = logits.T
    logits_t_ref[...] = lt

    l0 = lt[:_GROUP_SIZE]
    l1 = lt[_GROUP_SIZE:]
    m0 = jnp.max(l0, axis=0, keepdims=True)
    m1 = jnp.max(l1, axis=0, keepdims=True)
    row = jax.lax.broadcasted_iota(jnp.int32, l0.shape, 0)
    big = jnp.int32(_N_EXPERTS)
    i0 = jnp.min(jnp.where(l0 >= m0, row, big), axis=0, keepdims=True)
    i1 = jnp.min(jnp.where(l1 >= m1, row + _GROUP_SIZE, big),
                 axis=0, keepdims=True)
    s0 = jax.nn.sigmoid(m0)
    s1 = jax.nn.sigmoid(m1)
    inv = _ROUTED_SCALING / (s0 + s1 + 1e-10)
    idx_t_ref[...] = jnp.concatenate([i0, i1], axis=0)
    wgt_t_ref[...] = jnp.concatenate([s0 * inv, s1 * inv], axis=0)


def kernel(hidden_states, gate_weight):
    n, d = hidden_states.shape
    e = gate_weight.shape[0]
    logits_t, idx_t, wgt_t = pl.pallas_call(
        _gate_kernel,
        grid=(n // _BLOCK_N,),
        in_specs=[
            pl.BlockSpec((_BLOCK_N, d), lambda i: (i, 0)),
            pl.BlockSpec((e, d), lambda i: (0, 0)),
        ],
        out_specs=[
            pl.BlockSpec((e, _BLOCK_N), lambda i: (0, i)),
            pl.BlockSpec((_N_GROUP, _BLOCK_N), lambda i: (0, i)),
            pl.BlockSpec((_N_GROUP, _BLOCK_N), lambda i: (0, i)),
        ],
        out_shape=[
            jax.ShapeDtypeStruct((e, n), jnp.float32),
            jax.ShapeDtypeStruct((_N_GROUP, n), jnp.int32),
            jax.ShapeDtypeStruct((_N_GROUP, n), jnp.float32),
        ],
        compiler_params=pltpu.CompilerParams(
            dimension_semantics=("parallel",),
        ),
    )(hidden_states, gate_weight)
    return (idx_t.T, wgt_t.T, logits_t.T)
